# baseline (speedup 1.0000x reference)
import functools

import jax
import jax.numpy as jnp
import numpy as np
from jax import lax
from jax.experimental import pallas as pl
from jax.experimental.pallas import tpu as pltpu
from jax.experimental.pallas import tpu_sc as plsc

EMB = 32
BATCH = 16384
HIDDEN = 64
NROWS = 1000000
_PACK = 8
_TBLK = 8192
_TGRID = 16
_SPLIT8 = _TGRID * _TBLK
_QSHIFT = 17
_LASTBLK = (NROWS - 1) // _TBLK

_NC = 2
_NS = 16
_NW = _NC * _NS
_BPW = BATCH // _NW
_CHUNK = 128
_NCH = _BPW // _CHUNK

_HI = np.uint32(0xFFFF0000)


_RND = np.uint32(0x8000)


def _pack_pair(x):
    hi = (lax.bitcast_convert_type(x[:16], jnp.uint32) + _RND) & _HI
    lo = lax.shift_right_logical(
        lax.bitcast_convert_type(x[16:], jnp.uint32) + _RND, np.uint32(16))
    return lax.bitcast_convert_type(hi | lo, jnp.float32)


def _transpose_body(x0, x1, x2, x3, x4, x5, x6, x7, o_ref):
    xcat = jnp.concatenate(
        [_pack_pair(x[...]) for x in (x0, x1, x2, x3, x4, x5, x6, x7)],
        axis=0)
    o_ref[...] = xcat.T


def _pack_table(tabT):
    in_specs = [
        pl.BlockSpec(
            (EMB, _TBLK),
            functools.partial(
                lambda m, kk: (0, jnp.minimum(_TGRID * kk + m, _LASTBLK)),
                kk=k))
        for k in range(_PACK)
    ]
    return pl.pallas_call(
        _transpose_body,
        grid=(_TGRID,),
        in_specs=in_specs,
        out_specs=pl.BlockSpec((_TBLK, 128), lambda m: (m, 0)),
        out_shape=jax.ShapeDtypeStruct((_SPLIT8, 128), jnp.float32),
    )(*([tabT] * _PACK))


def _sc_gather_body(idx_hbm, tab_hbm, out_hbm, idx_v, rows_v, sem):
    wid = lax.axis_index("s") * _NC + lax.axis_index("c")
    base = wid * _BPW
    pltpu.sync_copy(idx_hbm.at[wid], idx_v)
    copies = [
        pltpu.async_copy(tab_hbm.at[idx_v.at[j]],
                         rows_v.at[pl.ds(j * _CHUNK, _CHUNK)], sem)
        for j in range(_NCH)
    ]
    for c in copies:
        c.wait()
    pltpu.sync_copy(rows_v, out_hbm.at[pl.ds(base, _BPW)])


_sc_gather = functools.partial(
    pl.kernel,
    out_type=jax.ShapeDtypeStruct((BATCH, 128), jnp.float32),
    mesh=plsc.VectorSubcoreMesh(core_axis_name="c", subcore_axis_name="s"),
    scratch_types=[
        pltpu.VMEM((_NCH, _CHUNK), jnp.int32),
        pltpu.VMEM((_BPW, 128), jnp.float32),
        pltpu.SemaphoreType.DMA,
    ],
)(_sc_gather_body)


_BM = 2048


def _unpack(x, m):
    p = lax.bitcast_convert_type(x, jnp.uint32) & m
    a = lax.bitcast_convert_type(p & _HI, jnp.float32)
    b = lax.bitcast_convert_type(
        lax.shift_left(p, np.uint32(16)), jnp.float32)
    return a, b


def _mlp_body(ug_ref, ig_ref, uq_ref, iq_ref, wua_ref, wub_ref, wia_ref,
              wib_ref, b1_ref, w2t_ref, b2_ref, o_ref):
    patt = lax.broadcasted_iota(jnp.int32, (1, 128), 1) // 16
    ones = np.uint32(0xFFFFFFFF)
    zero = np.uint32(0)
    mu = jnp.where(uq_ref[...] == patt, ones, zero)
    mi = jnp.where(iq_ref[...] == patt, ones, zero)
    ua, ub = _unpack(ug_ref[...], mu)
    ia, ib = _unpack(ig_ref[...], mi)
    h = (jnp.dot(ua, wua_ref[...], preferred_element_type=jnp.float32)
         + jnp.dot(ub, wub_ref[...], preferred_element_type=jnp.float32)
         + jnp.dot(ia, wia_ref[...], preferred_element_type=jnp.float32)
         + jnp.dot(ib, wib_ref[...], preferred_element_type=jnp.float32)
         + b1_ref[...])
    h = jnp.maximum(h, 0.0)
    o_ref[...] = (jnp.sum(h * w2t_ref[...], axis=1, keepdims=True)
                  + b2_ref[...])


def _mlp(u_grp, i_grp, uq, iq, wua, wub, wia, wib, b1, w2t, b2):
    grid = (BATCH // _BM,)
    wspec = pl.BlockSpec((128, HIDDEN), lambda m: (0, 0))
    return pl.pallas_call(
        _mlp_body,
        grid=grid,
        in_specs=[
            pl.BlockSpec((_BM, 128), lambda m: (m, 0)),
            pl.BlockSpec((_BM, 128), lambda m: (m, 0)),
            pl.BlockSpec((_BM, 1), lambda m: (m, 0)),
            pl.BlockSpec((_BM, 1), lambda m: (m, 0)),
            wspec, wspec, wspec, wspec,
            pl.BlockSpec((1, HIDDEN), lambda m: (0, 0)),
            pl.BlockSpec((1, HIDDEN), lambda m: (0, 0)),
            pl.BlockSpec((1, 1), lambda m: (0, 0)),
        ],
        out_specs=pl.BlockSpec((_BM, 1), lambda m: (m, 0)),
        out_shape=jax.ShapeDtypeStruct((BATCH, 1), jnp.float32),
    )(u_grp, i_grp, uq, iq, wua, wub, wia, wib, b1, w2t, b2)


@jax.jit
def kernel(user, item, user_table, item_table, W1, b1, W2, b2):
    user = user.astype(jnp.int32)
    item = item.astype(jnp.int32)
    ugidx = (user & (_SPLIT8 - 1)).reshape(_NW, _NCH, _CHUNK)
    igidx = (item & (_SPLIT8 - 1)).reshape(_NW, _NCH, _CHUNK)
    upk = _pack_table(user_table.T)
    u_grp = _sc_gather(ugidx, upk)
    ipk = _pack_table(item_table.T)
    i_grp = _sc_gather(igidx, ipk)
    uq = (user >> _QSHIFT).reshape(BATCH, 1)
    iq = (item >> _QSHIFT).reshape(BATCH, 1)
    wua = jnp.concatenate([W1[0:16]] * _PACK, axis=0)
    wub = jnp.concatenate([W1[16:32]] * _PACK, axis=0)
    wia = jnp.concatenate([W1[32:48]] * _PACK, axis=0)
    wib = jnp.concatenate([W1[48:64]] * _PACK, axis=0)
    return _mlp(u_grp, i_grp, uq, iq, wua, wub, wia, wib,
                b1.reshape(1, HIDDEN), W2.reshape(1, HIDDEN),
                b2.reshape(1, 1))


# speedup vs baseline: 1.0116x; 1.0116x over previous
import functools

import jax
import jax.numpy as jnp
import numpy as np
from jax import lax
from jax.experimental import pallas as pl
from jax.experimental.pallas import tpu as pltpu
from jax.experimental.pallas import tpu_sc as plsc

EMB = 32
BATCH = 16384
HIDDEN = 64
NROWS = 1000000
_PACK = 8
_TBLK = 16384
_TGRID = 8
_SPLIT8 = _TGRID * _TBLK
_QSHIFT = 17
_LASTBLK = (NROWS - 1) // _TBLK

_NC = 2
_NS = 16
_NW = _NC * _NS
_BPW = BATCH // _NW
_CHUNK = 128
_NCH = _BPW // _CHUNK

_HI = np.uint32(0xFFFF0000)


_RND = np.uint32(0x8000)


def _pack_pair(x):
    hi = (lax.bitcast_convert_type(x[:16], jnp.uint32) + _RND) & _HI
    lo = lax.shift_right_logical(
        lax.bitcast_convert_type(x[16:], jnp.uint32) + _RND, np.uint32(16))
    return lax.bitcast_convert_type(hi | lo, jnp.float32)


def _transpose_body(x0, x1, x2, x3, x4, x5, x6, x7, o_ref):
    xcat = jnp.concatenate(
        [_pack_pair(x[...]) for x in (x0, x1, x2, x3, x4, x5, x6, x7)],
        axis=0)
    o_ref[...] = xcat.T


def _pack_table(tabT):
    in_specs = [
        pl.BlockSpec(
            (EMB, _TBLK),
            functools.partial(
                lambda m, kk: (0, jnp.minimum(_TGRID * kk + m, _LASTBLK)),
                kk=k))
        for k in range(_PACK)
    ]
    return pl.pallas_call(
        _transpose_body,
        grid=(_TGRID,),
        in_specs=in_specs,
        out_specs=pl.BlockSpec((_TBLK, 128), lambda m: (m, 0)),
        out_shape=jax.ShapeDtypeStruct((_SPLIT8, 128), jnp.float32),
    )(*([tabT] * _PACK))


def _sc_gather_body(idx_hbm, tab_hbm, out_hbm, idx_v, rows_v, sem):
    wid = lax.axis_index("s") * _NC + lax.axis_index("c")
    base = wid * _BPW
    pltpu.sync_copy(idx_hbm.at[wid], idx_v)
    copies = [
        pltpu.async_copy(tab_hbm.at[idx_v.at[j]],
                         rows_v.at[pl.ds(j * _CHUNK, _CHUNK)], sem)
        for j in range(_NCH)
    ]
    for c in copies:
        c.wait()
    pltpu.sync_copy(rows_v, out_hbm.at[pl.ds(base, _BPW)])


_sc_gather = functools.partial(
    pl.kernel,
    out_type=jax.ShapeDtypeStruct((BATCH, 128), jnp.float32),
    mesh=plsc.VectorSubcoreMesh(core_axis_name="c", subcore_axis_name="s"),
    scratch_types=[
        pltpu.VMEM((_NCH, _CHUNK), jnp.int32),
        pltpu.VMEM((_BPW, 128), jnp.float32),
        pltpu.SemaphoreType.DMA,
    ],
)(_sc_gather_body)


_BM = 4096


def _unpack(x, m):
    p = lax.bitcast_convert_type(x, jnp.uint32) & m
    a = lax.bitcast_convert_type(p & _HI, jnp.float32)
    b = lax.bitcast_convert_type(
        lax.shift_left(p, np.uint32(16)), jnp.float32)
    return a, b


def _mlp_body(ug_ref, ig_ref, uq_ref, iq_ref, wua_ref, wub_ref, wia_ref,
              wib_ref, b1_ref, w2t_ref, b2_ref, o_ref):
    patt = lax.broadcasted_iota(jnp.int32, (1, 128), 1) // 16
    ones = np.uint32(0xFFFFFFFF)
    zero = np.uint32(0)
    mu = jnp.where(uq_ref[...] == patt, ones, zero)
    mi = jnp.where(iq_ref[...] == patt, ones, zero)
    ua, ub = _unpack(ug_ref[...], mu)
    ia, ib = _unpack(ig_ref[...], mi)
    h = (jnp.dot(ua, wua_ref[...], preferred_element_type=jnp.float32)
         + jnp.dot(ub, wub_ref[...], preferred_element_type=jnp.float32)
         + jnp.dot(ia, wia_ref[...], preferred_element_type=jnp.float32)
         + jnp.dot(ib, wib_ref[...], preferred_element_type=jnp.float32)
         + b1_ref[...])
    h = jnp.maximum(h, 0.0)
    o_ref[...] = (jnp.sum(h * w2t_ref[...], axis=1, keepdims=True)
                  + b2_ref[...])


def _mlp(u_grp, i_grp, uq, iq, wua, wub, wia, wib, b1, w2t, b2):
    grid = (BATCH // _BM,)
    wspec = pl.BlockSpec((128, HIDDEN), lambda m: (0, 0))
    return pl.pallas_call(
        _mlp_body,
        grid=grid,
        in_specs=[
            pl.BlockSpec((_BM, 128), lambda m: (m, 0)),
            pl.BlockSpec((_BM, 128), lambda m: (m, 0)),
            pl.BlockSpec((_BM, 1), lambda m: (m, 0)),
            pl.BlockSpec((_BM, 1), lambda m: (m, 0)),
            wspec, wspec, wspec, wspec,
            pl.BlockSpec((1, HIDDEN), lambda m: (0, 0)),
            pl.BlockSpec((1, HIDDEN), lambda m: (0, 0)),
            pl.BlockSpec((1, 1), lambda m: (0, 0)),
        ],
        out_specs=pl.BlockSpec((_BM, 1), lambda m: (m, 0)),
        out_shape=jax.ShapeDtypeStruct((BATCH, 1), jnp.float32),
    )(u_grp, i_grp, uq, iq, wua, wub, wia, wib, b1, w2t, b2)


@jax.jit
def kernel(user, item, user_table, item_table, W1, b1, W2, b2):
    user = user.astype(jnp.int32)
    item = item.astype(jnp.int32)
    ugidx = (user & (_SPLIT8 - 1)).reshape(_NW, _NCH, _CHUNK)
    igidx = (item & (_SPLIT8 - 1)).reshape(_NW, _NCH, _CHUNK)
    upk = _pack_table(user_table.T)
    u_grp = _sc_gather(ugidx, upk)
    ipk = _pack_table(item_table.T)
    i_grp = _sc_gather(igidx, ipk)
    uq = (user >> _QSHIFT).reshape(BATCH, 1)
    iq = (item >> _QSHIFT).reshape(BATCH, 1)
    wua = jnp.concatenate([W1[0:16]] * _PACK, axis=0)
    wub = jnp.concatenate([W1[16:32]] * _PACK, axis=0)
    wia = jnp.concatenate([W1[32:48]] * _PACK, axis=0)
    wib = jnp.concatenate([W1[48:64]] * _PACK, axis=0)
    return _mlp(u_grp, i_grp, uq, iq, wua, wub, wia, wib,
                b1.reshape(1, HIDDEN), W2.reshape(1, HIDDEN),
                b2.reshape(1, 1))
